# Initial kernel scaffold; baseline (speedup 1.0000x reference)
import jax
import jax.numpy as jnp
from jax.experimental import pallas as pl


def kernel(rna_ids, rna_feat, atac_ids, atac_feat, chrom_mask, rna_batch, atac_batch, params):
    raise NotImplementedError("write your pallas kernel here")


# speedup vs baseline: 10.6912x; 10.6912x over previous
import functools

import jax
import jax.numpy as jnp
from jax.experimental import pallas as pl
from jax.experimental.pallas import tpu as pltpu

NR, NA = 1024, 8192
FR = 192
FA = 192
ID_DIM = 64
HID = 256
HEADS = 4
HEAD_DIM = HID // HEADS
NGRAPH = 16
TOPK = 10
KPAD = 16


def _mm_kernel(x_ref, w_ref, b_ref, o_ref, *, act):
    acc = jax.lax.dot_general(
        x_ref[...], w_ref[...], (((1,), (0,)), ((), ())),
        preferred_element_type=jnp.float32)
    acc = acc + b_ref[...]
    if act == "relu":
        acc = jnp.maximum(acc, 0.0)
    o_ref[...] = acc


def _mm(x, w, b, act=None, bm=1024):
    m, k = x.shape
    n = w.shape[0]
    wt = w.T
    b2 = b.reshape(1, n)
    bm = min(bm, m)
    grid = (m // bm,)
    return pl.pallas_call(
        functools.partial(_mm_kernel, act=act),
        grid=grid,
        in_specs=[
            pl.BlockSpec((bm, k), lambda i: (i, 0)),
            pl.BlockSpec((k, n), lambda i: (0, 0)),
            pl.BlockSpec((1, n), lambda i: (0, 0)),
        ],
        out_specs=pl.BlockSpec((bm, n), lambda i: (i, 0)),
        out_shape=jax.ShapeDtypeStruct((m, n), jnp.float32),
    )(x, wt, b2)


def _topk_kernel(q_ref, k_ref, m_ref, w_ref, i_ref, s_ref, *, bn):
    s = jax.lax.dot_general(
        q_ref[0], k_ref[0], (((1,), (1,)), ((), ())),
        preferred_element_type=jnp.float32)
    s_ref[...] = s * m_ref[...]
    iota_c = jax.lax.broadcasted_iota(jnp.int32, (bn, NA), 1)
    lane16 = jax.lax.broadcasted_iota(jnp.int32, (bn, KPAD), 1)
    vals = jnp.full((bn, KPAD), -jnp.inf, jnp.float32)
    idxs = jnp.zeros((bn, KPAD), jnp.int32)
    for j in range(TOPK):
        s = s_ref[...]
        mx = jnp.max(s, axis=1, keepdims=True)
        am = jnp.min(jnp.where(s == mx, iota_c, NA), axis=1, keepdims=True)
        s_ref[...] = jnp.where(iota_c == am, -jnp.inf, s)
        vals = jnp.where(lane16 == j, mx, vals)
        idxs = jnp.where(lane16 == j, am, idxs)
    valid = lane16 < TOPK
    sg = jax.nn.sigmoid(vals)
    smax = jnp.max(jnp.where(valid, sg, -jnp.inf), axis=1, keepdims=True)
    e = jnp.where(valid, jnp.exp(sg - smax), 0.0)
    w = e / jnp.sum(e, axis=1, keepdims=True)
    w = jnp.where(valid & (vals > 0.0), w, 0.0)
    w_ref[0] = w
    i_ref[0] = idxs


def _topk(q, k, mask2d, bn=128):
    nb = NR // bn
    q3 = q.reshape(NR, HEADS, HEAD_DIM).transpose(1, 0, 2)
    k3 = k.reshape(NA, HEADS, HEAD_DIM).transpose(1, 0, 2)
    return pl.pallas_call(
        functools.partial(_topk_kernel, bn=bn),
        grid=(nb, HEADS),
        in_specs=[
            pl.BlockSpec((1, bn, HEAD_DIM), lambda b, h: (h, b, 0)),
            pl.BlockSpec((1, NA, HEAD_DIM), lambda b, h: (h, 0, 0)),
            pl.BlockSpec((bn, NA), lambda b, h: (b, 0)),
        ],
        out_specs=[
            pl.BlockSpec((1, bn, KPAD), lambda b, h: (h, b, 0)),
            pl.BlockSpec((1, bn, KPAD), lambda b, h: (h, b, 0)),
        ],
        out_shape=[
            jax.ShapeDtypeStruct((HEADS, NR, KPAD), jnp.float32),
            jax.ShapeDtypeStruct((HEADS, NR, KPAD), jnp.int32),
        ],
        scratch_shapes=[pltpu.VMEM((bn, NA), jnp.float32)],
    )(q3, k3, mask2d)


def _segsum_kernel(ids_ref, x_ref, sum_ref, cnt_ref):
    b = pl.program_id(0)

    @pl.when(b == 0)
    def _():
        sum_ref[...] = jnp.zeros_like(sum_ref)
        cnt_ref[...] = jnp.zeros_like(cnt_ref)

    seg = jax.lax.broadcasted_iota(jnp.int32, (NGRAPH, 1), 0)
    oh = (ids_ref[...] == seg).astype(jnp.float32)
    sum_ref[...] += jax.lax.dot_general(
        oh, x_ref[...], (((1,), (0,)), ((), ())),
        preferred_element_type=jnp.float32)
    cnt_ref[...] += jnp.broadcast_to(
        jnp.sum(oh, axis=1, keepdims=True), cnt_ref.shape)


def _segsum(x, ids, bn=1024):
    m, n = x.shape
    ids2 = ids.reshape(1, m).astype(jnp.int32)
    grid = (m // bn,)
    return pl.pallas_call(
        _segsum_kernel,
        grid=grid,
        in_specs=[
            pl.BlockSpec((1, bn), lambda i: (0, i)),
            pl.BlockSpec((bn, n), lambda i: (i, 0)),
        ],
        out_specs=[
            pl.BlockSpec((NGRAPH, n), lambda i: (0, 0)),
            pl.BlockSpec((NGRAPH, 128), lambda i: (0, 0)),
        ],
        out_shape=[
            jax.ShapeDtypeStruct((NGRAPH, n), jnp.float32),
            jax.ShapeDtypeStruct((NGRAPH, 128), jnp.float32),
        ],
    )(ids2, x)


def _mlp_kernel(sr_ref, cr_ref, sa_ref, ca_ref,
                w1_ref, b1_ref, w2_ref, b2_ref, w3_ref, b3_ref, o_ref):
    mean_r = sr_ref[...] / jnp.maximum(cr_ref[:, 0:1], 1.0)
    mean_a = sa_ref[...] / jnp.maximum(ca_ref[:, 0:1], 1.0)
    x = jnp.concatenate([mean_r, mean_a], axis=1)
    x = jax.lax.dot_general(x, w1_ref[...], (((1,), (0,)), ((), ())),
                            preferred_element_type=jnp.float32) + b1_ref[...]
    x = jnp.maximum(x, 0.0)
    x = jax.lax.dot_general(x, w2_ref[...], (((1,), (0,)), ((), ())),
                            preferred_element_type=jnp.float32) + b2_ref[...]
    x = jnp.maximum(x, 0.0)
    x = jax.lax.dot_general(x, w3_ref[...], (((1,), (0,)), ((), ())),
                            preferred_element_type=jnp.float32) + b3_ref[...]
    o_ref[...] = x


def _head_mlp(sum_r, cnt_r, sum_a, cnt_a, p):
    args = (sum_r, cnt_r, sum_a, cnt_a,
            p["fc1_w"].T, p["fc1_b"].reshape(1, -1),
            p["fc2_w"].T, p["fc2_b"].reshape(1, -1),
            p["fc3_w"].T, p["fc3_b"].reshape(1, -1))
    return pl.pallas_call(
        _mlp_kernel,
        out_shape=jax.ShapeDtypeStruct((NGRAPH, 2), jnp.float32),
    )(*args)


def kernel(rna_ids, rna_feat, atac_ids, atac_feat, chrom_mask,
           rna_batch, atac_batch, params):
    p = params
    x_rna = jnp.concatenate([p["rna_emb"][rna_ids], rna_feat], axis=1)
    x_atac = jnp.concatenate([p["atac_emb"][atac_ids], atac_feat], axis=1)
    mask2d = chrom_mask[..., 0]

    wr = jnp.concatenate([p["rna_query_w"], p["rna_value_w"], p["rna_self_w"]], 0)
    br = jnp.concatenate([p["rna_query_b"], p["rna_value_b"], p["rna_self_b"]], 0)
    wa = jnp.concatenate([p["atac_key_w"], p["atac_value_w"], p["atac_self_w"]], 0)
    ba = jnp.concatenate([p["atac_key_b"], p["atac_value_b"], p["atac_self_b"]], 0)
    yr = _mm(x_rna, wr, br)
    ya = _mm(x_atac, wa, ba)
    q, rv, rself = yr[:, 0:HID], yr[:, HID:2 * HID], yr[:, 2 * HID:]
    k, av, aself = ya[:, 0:HID], ya[:, HID:2 * HID], ya[:, 2 * HID:]

    w10, i10 = _topk(q, k, mask2d)

    av4 = av.reshape(NA, HEADS, HEAD_DIM)
    rv4 = rv.reshape(NR, HEADS, HEAD_DIM)
    fidx = i10 * HEADS + jnp.arange(HEADS, dtype=jnp.int32)[:, None, None]
    avf = av4.reshape(NA * HEADS, HEAD_DIM)
    gath = avf[fidx]
    r2a = jnp.einsum("hnj,hnjd->nhd", w10, gath).reshape(NR, HID)
    rv_h = jnp.transpose(rv4, (1, 0, 2))
    contrib = w10[..., None] * rv_h[:, :, None, :]
    a2rf = jax.ops.segment_sum(contrib.reshape(-1, HEAD_DIM),
                               fidx.reshape(-1), num_segments=NA * HEADS)
    a2r = a2rf.reshape(NA, HID)

    r2a_o = _mm(r2a, p["rna_out_w"], p["rna_out_b"])
    a2r_o = _mm(a2r, p["atac_out_w"], p["atac_out_b"])
    red_r = _mm(jnp.concatenate([r2a_o, rself], 1), p["red_rna_w"], p["red_rna_b"])
    red_a = _mm(jnp.concatenate([a2r_o, aself], 1), p["red_atac_w"], p["red_atac_b"])

    sum_r, cnt_r = _segsum(red_r, rna_batch)
    sum_a, cnt_a = _segsum(red_a, atac_batch)
    return _head_mlp(sum_r, cnt_r, sum_a, cnt_a, p)


# speedup vs baseline: 19.1530x; 1.7915x over previous
import functools

import jax
import jax.numpy as jnp
from jax.experimental import pallas as pl
from jax.experimental.pallas import tpu as pltpu

NR, NA = 1024, 8192
FR = 192
FA = 192
ID_DIM = 64
HID = 256
HEADS = 4
HEAD_DIM = HID // HEADS
NGRAPH = 16
TOPK = 10
KPAD = 16


def _mm_kernel(x_ref, w_ref, b_ref, o_ref, *, act):
    acc = jax.lax.dot_general(
        x_ref[...], w_ref[...], (((1,), (0,)), ((), ())),
        preferred_element_type=jnp.float32)
    acc = acc + b_ref[...]
    if act == "relu":
        acc = jnp.maximum(acc, 0.0)
    o_ref[...] = acc


def _mm(x, w, b, act=None, bm=1024):
    m, k = x.shape
    n = w.shape[0]
    wt = w.T
    b2 = b.reshape(1, n)
    bm = min(bm, m)
    grid = (m // bm,)
    return pl.pallas_call(
        functools.partial(_mm_kernel, act=act),
        grid=grid,
        in_specs=[
            pl.BlockSpec((bm, k), lambda i: (i, 0)),
            pl.BlockSpec((k, n), lambda i: (0, 0)),
            pl.BlockSpec((1, n), lambda i: (0, 0)),
        ],
        out_specs=pl.BlockSpec((bm, n), lambda i: (i, 0)),
        out_shape=jax.ShapeDtypeStruct((m, n), jnp.float32),
    )(x, wt, b2)


def _attn_kernel(q_ref, k_ref, m_ref, av_ref, rv_ref, r2a_ref, a2r_ref,
                 s0_ref, s1_ref, *, bn, nb):
    b = pl.program_id(0)
    h = pl.program_id(1)

    @pl.when((b == 0) & (h == 0))
    def _():
        a2r_ref[...] = jnp.zeros_like(a2r_ref)

    s = jax.lax.dot_general(
        q_ref[0], k_ref[0], (((1,), (1,)), ((), ())),
        preferred_element_type=jnp.float32)
    s = s * m_ref[...]
    s0_ref[...] = s
    s1_ref[...] = s
    lane16 = jax.lax.broadcasted_iota(jnp.int32, (bn, KPAD), 1)
    vals = jnp.full((bn, KPAD), -jnp.inf, jnp.float32)
    for j in range(TOPK):
        s = s1_ref[...]
        mx = jnp.max(s, axis=1, keepdims=True)
        s1_ref[...] = jnp.where(s == mx, -jnp.inf, s)
        vals = jnp.where(lane16 == j, mx, vals)
    sg = jax.nn.sigmoid(vals)
    smax = sg[:, 0:1]
    e = jnp.where(lane16 < TOPK, jnp.exp(sg - smax), 0.0)
    z = jnp.sum(e, axis=1, keepdims=True)
    s0 = s0_ref[...]
    s1 = s1_ref[...]
    attnw = jnp.where(
        (s1 == -jnp.inf) & (s0 > 0.0),
        jnp.exp(jax.nn.sigmoid(s0) - smax) / z, 0.0)
    r2a_ref[0] = jax.lax.dot_general(
        attnw, av_ref[0], (((1,), (0,)), ((), ())),
        preferred_element_type=jnp.float32)
    a2r_ref[h] += jax.lax.dot_general(
        attnw, rv_ref[0], (((0,), (0,)), ((), ())),
        preferred_element_type=jnp.float32)


def _attn(q, k, av, rv, mask2d, bn=128):
    nb = NR // bn
    q3 = q.reshape(NR, HEADS, HEAD_DIM).transpose(1, 0, 2)
    k3 = k.reshape(NA, HEADS, HEAD_DIM).transpose(1, 0, 2)
    av3 = av.reshape(NA, HEADS, HEAD_DIM).transpose(1, 0, 2)
    rv3 = rv.reshape(NR, HEADS, HEAD_DIM).transpose(1, 0, 2)
    r2a3, a2r3 = pl.pallas_call(
        functools.partial(_attn_kernel, bn=bn, nb=nb),
        grid=(nb, HEADS),
        in_specs=[
            pl.BlockSpec((1, bn, HEAD_DIM), lambda b, h: (h, b, 0)),
            pl.BlockSpec((1, NA, HEAD_DIM), lambda b, h: (h, 0, 0)),
            pl.BlockSpec((bn, NA), lambda b, h: (b, 0)),
            pl.BlockSpec((1, NA, HEAD_DIM), lambda b, h: (h, 0, 0)),
            pl.BlockSpec((1, bn, HEAD_DIM), lambda b, h: (h, b, 0)),
        ],
        out_specs=[
            pl.BlockSpec((1, bn, HEAD_DIM), lambda b, h: (h, b, 0)),
            pl.BlockSpec((HEADS, NA, HEAD_DIM), lambda b, h: (0, 0, 0)),
        ],
        out_shape=[
            jax.ShapeDtypeStruct((HEADS, NR, HEAD_DIM), jnp.float32),
            jax.ShapeDtypeStruct((HEADS, NA, HEAD_DIM), jnp.float32),
        ],
        scratch_shapes=[pltpu.VMEM((bn, NA), jnp.float32),
                        pltpu.VMEM((bn, NA), jnp.float32)],
    )(q3, k3, mask2d, av3, rv3)
    r2a = r2a3.transpose(1, 0, 2).reshape(NR, HID)
    a2r = a2r3.transpose(1, 0, 2).reshape(NA, HID)
    return r2a, a2r


def _segsum_kernel(ids_ref, x_ref, sum_ref, cnt_ref):
    b = pl.program_id(0)

    @pl.when(b == 0)
    def _():
        sum_ref[...] = jnp.zeros_like(sum_ref)
        cnt_ref[...] = jnp.zeros_like(cnt_ref)

    seg = jax.lax.broadcasted_iota(jnp.int32, (NGRAPH, 1), 0)
    oh = (ids_ref[...] == seg).astype(jnp.float32)
    sum_ref[...] += jax.lax.dot_general(
        oh, x_ref[...], (((1,), (0,)), ((), ())),
        preferred_element_type=jnp.float32)
    cnt_ref[...] += jnp.broadcast_to(
        jnp.sum(oh, axis=1, keepdims=True), cnt_ref.shape)


def _segsum(x, ids, bn=1024):
    m, n = x.shape
    ids2 = ids.reshape(1, m).astype(jnp.int32)
    grid = (m // bn,)
    return pl.pallas_call(
        _segsum_kernel,
        grid=grid,
        in_specs=[
            pl.BlockSpec((1, bn), lambda i: (0, i)),
            pl.BlockSpec((bn, n), lambda i: (i, 0)),
        ],
        out_specs=[
            pl.BlockSpec((NGRAPH, n), lambda i: (0, 0)),
            pl.BlockSpec((NGRAPH, 128), lambda i: (0, 0)),
        ],
        out_shape=[
            jax.ShapeDtypeStruct((NGRAPH, n), jnp.float32),
            jax.ShapeDtypeStruct((NGRAPH, 128), jnp.float32),
        ],
    )(ids2, x)


def _mlp_kernel(sr_ref, cr_ref, sa_ref, ca_ref,
                w1_ref, b1_ref, w2_ref, b2_ref, w3_ref, b3_ref, o_ref):
    mean_r = sr_ref[...] / jnp.maximum(cr_ref[:, 0:1], 1.0)
    mean_a = sa_ref[...] / jnp.maximum(ca_ref[:, 0:1], 1.0)
    x = jnp.concatenate([mean_r, mean_a], axis=1)
    x = jax.lax.dot_general(x, w1_ref[...], (((1,), (0,)), ((), ())),
                            preferred_element_type=jnp.float32) + b1_ref[...]
    x = jnp.maximum(x, 0.0)
    x = jax.lax.dot_general(x, w2_ref[...], (((1,), (0,)), ((), ())),
                            preferred_element_type=jnp.float32) + b2_ref[...]
    x = jnp.maximum(x, 0.0)
    x = jax.lax.dot_general(x, w3_ref[...], (((1,), (0,)), ((), ())),
                            preferred_element_type=jnp.float32) + b3_ref[...]
    o_ref[...] = x


def _head_mlp(sum_r, cnt_r, sum_a, cnt_a, p):
    args = (sum_r, cnt_r, sum_a, cnt_a,
            p["fc1_w"].T, p["fc1_b"].reshape(1, -1),
            p["fc2_w"].T, p["fc2_b"].reshape(1, -1),
            p["fc3_w"].T, p["fc3_b"].reshape(1, -1))
    return pl.pallas_call(
        _mlp_kernel,
        out_shape=jax.ShapeDtypeStruct((NGRAPH, 2), jnp.float32),
    )(*args)


def kernel(rna_ids, rna_feat, atac_ids, atac_feat, chrom_mask,
           rna_batch, atac_batch, params):
    p = params
    x_rna = jnp.concatenate([p["rna_emb"][rna_ids], rna_feat], axis=1)
    x_atac = jnp.concatenate([p["atac_emb"][atac_ids], atac_feat], axis=1)
    mask2d = chrom_mask[..., 0]

    wr = jnp.concatenate([p["rna_query_w"], p["rna_value_w"], p["rna_self_w"]], 0)
    br = jnp.concatenate([p["rna_query_b"], p["rna_value_b"], p["rna_self_b"]], 0)
    wa = jnp.concatenate([p["atac_key_w"], p["atac_value_w"], p["atac_self_w"]], 0)
    ba = jnp.concatenate([p["atac_key_b"], p["atac_value_b"], p["atac_self_b"]], 0)
    yr = _mm(x_rna, wr, br)
    ya = _mm(x_atac, wa, ba)
    q, rv, rself = yr[:, 0:HID], yr[:, HID:2 * HID], yr[:, 2 * HID:]
    k, av, aself = ya[:, 0:HID], ya[:, HID:2 * HID], ya[:, 2 * HID:]

    r2a, a2r = _attn(q, k, av, rv, mask2d)

    r2a_o = _mm(r2a, p["rna_out_w"], p["rna_out_b"])
    a2r_o = _mm(a2r, p["atac_out_w"], p["atac_out_b"])
    red_r = _mm(jnp.concatenate([r2a_o, rself], 1), p["red_rna_w"], p["red_rna_b"])
    red_a = _mm(jnp.concatenate([a2r_o, aself], 1), p["red_atac_w"], p["red_atac_b"])

    sum_r, cnt_r = _segsum(red_r, rna_batch)
    sum_a, cnt_a = _segsum(red_a, atac_batch)
    return _head_mlp(sum_r, cnt_r, sum_a, cnt_a, p)


# speedup vs baseline: 22.7366x; 1.1871x over previous
import functools

import jax
import jax.numpy as jnp
from jax.experimental import pallas as pl
from jax.experimental.pallas import tpu as pltpu

NR, NA = 1024, 8192
FR = 192
ID_DIM = 64
HID = 256
HEADS = 4
HEAD_DIM = HID // HEADS
NGRAPH = 16
TOPK = 10
KPAD = 16


def _dot(a, b):
    return jax.lax.dot_general(a, b, (((1,), (0,)), ((), ())),
                               preferred_element_type=jnp.float32)


def _proj_kernel(xe_ref, xf_ref, wq_ref, bq_ref, wv_ref, bv_ref,
                 ws_ref, bs_ref, qo_ref, vo_ref, so_ref):
    h = pl.program_id(1)
    xe = xe_ref[...]
    xf = xf_ref[...]
    wq = wq_ref[0]
    qo_ref[0] = _dot(xe, wq[0:ID_DIM]) + _dot(xf, wq[ID_DIM:]) + bq_ref[0]
    wv = wv_ref[0]
    vo_ref[0] = _dot(xe, wv[0:ID_DIM]) + _dot(xf, wv[ID_DIM:]) + bv_ref[0]

    @pl.when(h == 0)
    def _():
        ws = ws_ref[...]
        so_ref[...] = _dot(xe, ws[0:ID_DIM]) + _dot(xf, ws[ID_DIM:]) + bs_ref[...]


def _proj_side(emb, feat, wq, bq, wv, bv, wself, bself, bm):
    n, cin = emb.shape[0], ID_DIM + FR
    wq3 = wq.T.reshape(cin, HEADS, HEAD_DIM).transpose(1, 0, 2)
    wv3 = wv.T.reshape(cin, HEADS, HEAD_DIM).transpose(1, 0, 2)
    bq3 = bq.reshape(HEADS, 1, HEAD_DIM)
    bv3 = bv.reshape(HEADS, 1, HEAD_DIM)
    nb = n // bm
    return pl.pallas_call(
        _proj_kernel,
        grid=(nb, HEADS),
        in_specs=[
            pl.BlockSpec((bm, ID_DIM), lambda b, h: (b, 0)),
            pl.BlockSpec((bm, FR), lambda b, h: (b, 0)),
            pl.BlockSpec((1, cin, HEAD_DIM), lambda b, h: (h, 0, 0)),
            pl.BlockSpec((1, 1, HEAD_DIM), lambda b, h: (h, 0, 0)),
            pl.BlockSpec((1, cin, HEAD_DIM), lambda b, h: (h, 0, 0)),
            pl.BlockSpec((1, 1, HEAD_DIM), lambda b, h: (h, 0, 0)),
            pl.BlockSpec((cin, HID), lambda b, h: (0, 0)),
            pl.BlockSpec((1, HID), lambda b, h: (0, 0)),
        ],
        out_specs=[
            pl.BlockSpec((1, bm, HEAD_DIM), lambda b, h: (h, b, 0)),
            pl.BlockSpec((1, bm, HEAD_DIM), lambda b, h: (h, b, 0)),
            pl.BlockSpec((bm, HID), lambda b, h: (b, 0)),
        ],
        out_shape=[
            jax.ShapeDtypeStruct((HEADS, n, HEAD_DIM), jnp.float32),
            jax.ShapeDtypeStruct((HEADS, n, HEAD_DIM), jnp.float32),
            jax.ShapeDtypeStruct((n, HID), jnp.float32),
        ],
    )(emb, feat, wq3, bq3, wv3, bv3, wself.T, bself.reshape(1, HID))


def _attn_kernel(q_ref, k_ref, m_ref, av_ref, rv_ref, r2a_ref, a2r_ref,
                 s0_ref, s1_ref, *, bn):
    b = pl.program_id(0)
    h = pl.program_id(1)

    @pl.when((b == 0) & (h == 0))
    def _():
        a2r_ref[...] = jnp.zeros_like(a2r_ref)

    s = jax.lax.dot_general(
        q_ref[0], k_ref[0], (((1,), (1,)), ((), ())),
        preferred_element_type=jnp.float32)
    s = s * m_ref[...]
    s0_ref[...] = s
    s1_ref[...] = s
    lane16 = jax.lax.broadcasted_iota(jnp.int32, (bn, KPAD), 1)
    vals = jnp.full((bn, KPAD), -jnp.inf, jnp.float32)
    for j in range(TOPK):
        s = s1_ref[...]
        mx = jnp.max(s, axis=1, keepdims=True)
        s1_ref[...] = jnp.where(s == mx, -jnp.inf, s)
        vals = jnp.where(lane16 == j, mx, vals)
    sg = jax.nn.sigmoid(vals)
    smax = sg[:, 0:1]
    e = jnp.where(lane16 < TOPK, jnp.exp(sg - smax), 0.0)
    z = jnp.sum(e, axis=1, keepdims=True)
    s0 = s0_ref[...]
    s1 = s1_ref[...]
    attnw = jnp.where(
        (s1 == -jnp.inf) & (s0 > 0.0),
        jnp.exp(jax.nn.sigmoid(s0) - smax) / z, 0.0)
    r2a_ref[0] = jax.lax.dot_general(
        attnw, av_ref[0], (((1,), (0,)), ((), ())),
        preferred_element_type=jnp.float32)
    a2r_ref[h] += jax.lax.dot_general(
        attnw, rv_ref[0], (((0,), (0,)), ((), ())),
        preferred_element_type=jnp.float32)


def _attn(q3, k3, av3, rv3, mask2d, bn=128):
    nb = NR // bn
    return pl.pallas_call(
        functools.partial(_attn_kernel, bn=bn),
        grid=(nb, HEADS),
        in_specs=[
            pl.BlockSpec((1, bn, HEAD_DIM), lambda b, h: (h, b, 0)),
            pl.BlockSpec((1, NA, HEAD_DIM), lambda b, h: (h, 0, 0)),
            pl.BlockSpec((bn, NA), lambda b, h: (b, 0)),
            pl.BlockSpec((1, NA, HEAD_DIM), lambda b, h: (h, 0, 0)),
            pl.BlockSpec((1, bn, HEAD_DIM), lambda b, h: (h, b, 0)),
        ],
        out_specs=[
            pl.BlockSpec((1, bn, HEAD_DIM), lambda b, h: (h, b, 0)),
            pl.BlockSpec((HEADS, NA, HEAD_DIM), lambda b, h: (0, 0, 0)),
        ],
        out_shape=[
            jax.ShapeDtypeStruct((HEADS, NR, HEAD_DIM), jnp.float32),
            jax.ShapeDtypeStruct((HEADS, NA, HEAD_DIM), jnp.float32),
        ],
        scratch_shapes=[pltpu.VMEM((bn, NA), jnp.float32),
                        pltpu.VMEM((bn, NA), jnp.float32)],
    )(q3, k3, mask2d, av3, rv3)


def _reduce_kernel(x3_ref, sf_ref, ids_ref, wo_ref, bo_ref,
                   rw1_ref, rw2_ref, rb_ref, sum_ref, cnt_ref):
    m = pl.program_id(0)

    @pl.when(m == 0)
    def _():
        sum_ref[...] = jnp.zeros_like(sum_ref)
        cnt_ref[...] = jnp.zeros_like(cnt_ref)

    tmp = bo_ref[...]
    for h in range(HEADS):
        tmp = tmp + _dot(x3_ref[h], wo_ref[h])
    red = _dot(tmp, rw1_ref[...]) + _dot(sf_ref[...], rw2_ref[...]) + rb_ref[...]
    seg = jax.lax.broadcasted_iota(jnp.int32, (NGRAPH, 1), 0)
    oh = (ids_ref[...] == seg).astype(jnp.float32)
    sum_ref[...] += jax.lax.dot_general(
        oh, red, (((1,), (0,)), ((), ())), preferred_element_type=jnp.float32)
    cnt_ref[...] += jnp.broadcast_to(
        jnp.sum(oh, axis=1, keepdims=True), cnt_ref.shape)


def _reduce_side(x3, selfp, ids, wo, bo, rw, rb, bm):
    n = selfp.shape[0]
    nb = n // bm
    wo3 = wo.T.reshape(HEADS, HEAD_DIM, HID)
    rwt = rw.T
    return pl.pallas_call(
        _reduce_kernel,
        grid=(nb,),
        in_specs=[
            pl.BlockSpec((HEADS, bm, HEAD_DIM), lambda m: (0, m, 0)),
            pl.BlockSpec((bm, HID), lambda m: (m, 0)),
            pl.BlockSpec((1, bm), lambda m: (0, m)),
            pl.BlockSpec((HEADS, HEAD_DIM, HID), lambda m: (0, 0, 0)),
            pl.BlockSpec((1, HID), lambda m: (0, 0)),
            pl.BlockSpec((HID, HID), lambda m: (0, 0)),
            pl.BlockSpec((HID, HID), lambda m: (0, 0)),
            pl.BlockSpec((1, HID), lambda m: (0, 0)),
        ],
        out_specs=[
            pl.BlockSpec((NGRAPH, HID), lambda m: (0, 0)),
            pl.BlockSpec((NGRAPH, 128), lambda m: (0, 0)),
        ],
        out_shape=[
            jax.ShapeDtypeStruct((NGRAPH, HID), jnp.float32),
            jax.ShapeDtypeStruct((NGRAPH, 128), jnp.float32),
        ],
    )(x3, selfp, ids.reshape(1, n).astype(jnp.int32), wo3,
      bo.reshape(1, HID), rwt[:HID], rwt[HID:], rb.reshape(1, HID))


def _mlp_kernel(sr_ref, cr_ref, sa_ref, ca_ref,
                w1_ref, b1_ref, w2_ref, b2_ref, w3_ref, b3_ref, o_ref):
    mean_r = sr_ref[...] / jnp.maximum(cr_ref[:, 0:1], 1.0)
    mean_a = sa_ref[...] / jnp.maximum(ca_ref[:, 0:1], 1.0)
    x = jnp.concatenate([mean_r, mean_a], axis=1)
    x = jnp.maximum(_dot(x, w1_ref[...]) + b1_ref[...], 0.0)
    x = jnp.maximum(_dot(x, w2_ref[...]) + b2_ref[...], 0.0)
    o_ref[...] = _dot(x, w3_ref[...]) + b3_ref[...]


def _head_mlp(sum_r, cnt_r, sum_a, cnt_a, p):
    args = (sum_r, cnt_r, sum_a, cnt_a,
            p["fc1_w"].T, p["fc1_b"].reshape(1, -1),
            p["fc2_w"].T, p["fc2_b"].reshape(1, -1),
            p["fc3_w"].T, p["fc3_b"].reshape(1, -1))
    return pl.pallas_call(
        _mlp_kernel,
        out_shape=jax.ShapeDtypeStruct((NGRAPH, 2), jnp.float32),
    )(*args)


def kernel(rna_ids, rna_feat, atac_ids, atac_feat, chrom_mask,
           rna_batch, atac_batch, params):
    p = params
    emb_r = p["rna_emb"][rna_ids]
    emb_a = p["atac_emb"][atac_ids]
    mask2d = chrom_mask[..., 0]

    q3, rv3, rself = _proj_side(
        emb_r, rna_feat, p["rna_query_w"], p["rna_query_b"],
        p["rna_value_w"], p["rna_value_b"],
        p["rna_self_w"], p["rna_self_b"], bm=1024)
    k3, av3, aself = _proj_side(
        emb_a, atac_feat, p["atac_key_w"], p["atac_key_b"],
        p["atac_value_w"], p["atac_value_b"],
        p["atac_self_w"], p["atac_self_b"], bm=1024)

    r2a3, a2r3 = _attn(q3, k3, av3, rv3, mask2d)

    sum_r, cnt_r = _reduce_side(r2a3, rself, rna_batch,
                                p["rna_out_w"], p["rna_out_b"],
                                p["red_rna_w"], p["red_rna_b"], bm=1024)
    sum_a, cnt_a = _reduce_side(a2r3, aself, atac_batch,
                                p["atac_out_w"], p["atac_out_b"],
                                p["red_atac_w"], p["red_atac_b"], bm=1024)
    return _head_mlp(sum_r, cnt_r, sum_a, cnt_a, p)


# speedup vs baseline: 23.1350x; 1.0175x over previous
import functools

import jax
import jax.numpy as jnp
from jax.experimental import pallas as pl
from jax.experimental.pallas import tpu as pltpu

NR, NA = 1024, 8192
FR = 192
ID_DIM = 64
HID = 256
HEADS = 4
HEAD_DIM = HID // HEADS
NGRAPH = 16
TOPK = 10
KPAD = 16


def _dot(a, b):
    return jax.lax.dot_general(a, b, (((1,), (0,)), ((), ())),
                               preferred_element_type=jnp.float32)


def _proj_kernel(xe_ref, xf_ref, wq_ref, bq_ref, wv_ref, bv_ref,
                 ws_ref, bs_ref, qo_ref, vo_ref, so_ref):
    h = pl.program_id(1)
    xe = xe_ref[...]
    xf = xf_ref[...]
    wq = wq_ref[0]
    qo_ref[0] = _dot(xe, wq[0:ID_DIM]) + _dot(xf, wq[ID_DIM:]) + bq_ref[0]
    wv = wv_ref[0]
    vo_ref[0] = _dot(xe, wv[0:ID_DIM]) + _dot(xf, wv[ID_DIM:]) + bv_ref[0]

    @pl.when(h == 0)
    def _():
        ws = ws_ref[...]
        so_ref[...] = _dot(xe, ws[0:ID_DIM]) + _dot(xf, ws[ID_DIM:]) + bs_ref[...]


def _proj_side(emb, feat, wq, bq, wv, bv, wself, bself, bm):
    n, cin = emb.shape[0], ID_DIM + FR
    wq3 = wq.T.reshape(cin, HEADS, HEAD_DIM).transpose(1, 0, 2)
    wv3 = wv.T.reshape(cin, HEADS, HEAD_DIM).transpose(1, 0, 2)
    bq3 = bq.reshape(HEADS, 1, HEAD_DIM)
    bv3 = bv.reshape(HEADS, 1, HEAD_DIM)
    nb = n // bm
    return pl.pallas_call(
        _proj_kernel,
        grid=(nb, HEADS),
        in_specs=[
            pl.BlockSpec((bm, ID_DIM), lambda b, h: (b, 0)),
            pl.BlockSpec((bm, FR), lambda b, h: (b, 0)),
            pl.BlockSpec((1, cin, HEAD_DIM), lambda b, h: (h, 0, 0)),
            pl.BlockSpec((1, 1, HEAD_DIM), lambda b, h: (h, 0, 0)),
            pl.BlockSpec((1, cin, HEAD_DIM), lambda b, h: (h, 0, 0)),
            pl.BlockSpec((1, 1, HEAD_DIM), lambda b, h: (h, 0, 0)),
            pl.BlockSpec((cin, HID), lambda b, h: (0, 0)),
            pl.BlockSpec((1, HID), lambda b, h: (0, 0)),
        ],
        out_specs=[
            pl.BlockSpec((1, bm, HEAD_DIM), lambda b, h: (h, b, 0)),
            pl.BlockSpec((1, bm, HEAD_DIM), lambda b, h: (h, b, 0)),
            pl.BlockSpec((bm, HID), lambda b, h: (b, 0)),
        ],
        out_shape=[
            jax.ShapeDtypeStruct((HEADS, n, HEAD_DIM), jnp.float32),
            jax.ShapeDtypeStruct((HEADS, n, HEAD_DIM), jnp.float32),
            jax.ShapeDtypeStruct((n, HID), jnp.float32),
        ],
    )(emb, feat, wq3, bq3, wv3, bv3, wself.T, bself.reshape(1, HID))


def _attn_kernel(q_ref, k_ref, m_ref, av_ref, rv_ref, r2a_ref, a2r_ref,
                 s0_ref, *, bn):
    b = pl.program_id(0)
    h = pl.program_id(1)

    @pl.when((b == 0) & (h == 0))
    def _():
        a2r_ref[...] = jnp.zeros_like(a2r_ref)

    s = jax.lax.dot_general(
        q_ref[0], k_ref[0], (((1,), (1,)), ((), ())),
        preferred_element_type=jnp.float32)
    s0_ref[...] = s * m_ref[...]
    lane16 = jax.lax.broadcasted_iota(jnp.int32, (bn, KPAD), 1)
    s = s0_ref[...]
    mx = jnp.max(s, axis=1, keepdims=True)
    vals = jnp.where(lane16 == 0, mx, -jnp.inf)
    for j in range(1, TOPK):
        s = s0_ref[...]
        mx = jnp.max(jnp.where(s < mx, s, -jnp.inf), axis=1, keepdims=True)
        vals = jnp.where(lane16 == j, mx, vals)
    v10 = mx
    sg = jax.nn.sigmoid(vals)
    smax = sg[:, 0:1]
    e = jnp.where(lane16 < TOPK, jnp.exp(sg - smax), 0.0)
    rz = 1.0 / jnp.sum(e, axis=1, keepdims=True)
    s0 = s0_ref[...]
    attnw = jnp.where(
        (s0 >= v10) & (s0 > 0.0),
        jnp.exp(jax.nn.sigmoid(s0) - smax) * rz, 0.0)
    r2a_ref[0] = jax.lax.dot_general(
        attnw, av_ref[0], (((1,), (0,)), ((), ())),
        preferred_element_type=jnp.float32)
    a2r_ref[h] += jax.lax.dot_general(
        attnw, rv_ref[0], (((0,), (0,)), ((), ())),
        preferred_element_type=jnp.float32)


def _attn(q3, k3, av3, rv3, mask2d, bn=128):
    nb = NR // bn
    return pl.pallas_call(
        functools.partial(_attn_kernel, bn=bn),
        grid=(nb, HEADS),
        in_specs=[
            pl.BlockSpec((1, bn, HEAD_DIM), lambda b, h: (h, b, 0)),
            pl.BlockSpec((1, NA, HEAD_DIM), lambda b, h: (h, 0, 0)),
            pl.BlockSpec((bn, NA), lambda b, h: (b, 0)),
            pl.BlockSpec((1, NA, HEAD_DIM), lambda b, h: (h, 0, 0)),
            pl.BlockSpec((1, bn, HEAD_DIM), lambda b, h: (h, b, 0)),
        ],
        out_specs=[
            pl.BlockSpec((1, bn, HEAD_DIM), lambda b, h: (h, b, 0)),
            pl.BlockSpec((HEADS, NA, HEAD_DIM), lambda b, h: (0, 0, 0)),
        ],
        out_shape=[
            jax.ShapeDtypeStruct((HEADS, NR, HEAD_DIM), jnp.float32),
            jax.ShapeDtypeStruct((HEADS, NA, HEAD_DIM), jnp.float32),
        ],
        scratch_shapes=[pltpu.VMEM((bn, NA), jnp.float32)],
    )(q3, k3, mask2d, av3, rv3)


def _reduce_kernel(x3_ref, sf_ref, ids_ref, wo_ref, bo_ref,
                   rw1_ref, rw2_ref, rb_ref, sum_ref, cnt_ref):
    m = pl.program_id(0)

    @pl.when(m == 0)
    def _():
        sum_ref[...] = jnp.zeros_like(sum_ref)
        cnt_ref[...] = jnp.zeros_like(cnt_ref)

    tmp = bo_ref[...]
    for h in range(HEADS):
        tmp = tmp + _dot(x3_ref[h], wo_ref[h])
    red = _dot(tmp, rw1_ref[...]) + _dot(sf_ref[...], rw2_ref[...]) + rb_ref[...]
    seg = jax.lax.broadcasted_iota(jnp.int32, (NGRAPH, 1), 0)
    oh = (ids_ref[...] == seg).astype(jnp.float32)
    sum_ref[...] += jax.lax.dot_general(
        oh, red, (((1,), (0,)), ((), ())), preferred_element_type=jnp.float32)
    cnt_ref[...] += jnp.broadcast_to(
        jnp.sum(oh, axis=1, keepdims=True), cnt_ref.shape)


def _reduce_side(x3, selfp, ids, wo, bo, rw, rb, bm):
    n = selfp.shape[0]
    nb = n // bm
    wo3 = wo.T.reshape(HEADS, HEAD_DIM, HID)
    rwt = rw.T
    return pl.pallas_call(
        _reduce_kernel,
        grid=(nb,),
        in_specs=[
            pl.BlockSpec((HEADS, bm, HEAD_DIM), lambda m: (0, m, 0)),
            pl.BlockSpec((bm, HID), lambda m: (m, 0)),
            pl.BlockSpec((1, bm), lambda m: (0, m)),
            pl.BlockSpec((HEADS, HEAD_DIM, HID), lambda m: (0, 0, 0)),
            pl.BlockSpec((1, HID), lambda m: (0, 0)),
            pl.BlockSpec((HID, HID), lambda m: (0, 0)),
            pl.BlockSpec((HID, HID), lambda m: (0, 0)),
            pl.BlockSpec((1, HID), lambda m: (0, 0)),
        ],
        out_specs=[
            pl.BlockSpec((NGRAPH, HID), lambda m: (0, 0)),
            pl.BlockSpec((NGRAPH, 128), lambda m: (0, 0)),
        ],
        out_shape=[
            jax.ShapeDtypeStruct((NGRAPH, HID), jnp.float32),
            jax.ShapeDtypeStruct((NGRAPH, 128), jnp.float32),
        ],
    )(x3, selfp, ids.reshape(1, n).astype(jnp.int32), wo3,
      bo.reshape(1, HID), rwt[:HID], rwt[HID:], rb.reshape(1, HID))


def _mlp_kernel(sr_ref, cr_ref, sa_ref, ca_ref,
                w1_ref, b1_ref, w2_ref, b2_ref, w3_ref, b3_ref, o_ref):
    mean_r = sr_ref[...] / jnp.maximum(cr_ref[:, 0:1], 1.0)
    mean_a = sa_ref[...] / jnp.maximum(ca_ref[:, 0:1], 1.0)
    x = jnp.concatenate([mean_r, mean_a], axis=1)
    x = jnp.maximum(_dot(x, w1_ref[...]) + b1_ref[...], 0.0)
    x = jnp.maximum(_dot(x, w2_ref[...]) + b2_ref[...], 0.0)
    o_ref[...] = _dot(x, w3_ref[...]) + b3_ref[...]


def _head_mlp(sum_r, cnt_r, sum_a, cnt_a, p):
    args = (sum_r, cnt_r, sum_a, cnt_a,
            p["fc1_w"].T, p["fc1_b"].reshape(1, -1),
            p["fc2_w"].T, p["fc2_b"].reshape(1, -1),
            p["fc3_w"].T, p["fc3_b"].reshape(1, -1))
    return pl.pallas_call(
        _mlp_kernel,
        out_shape=jax.ShapeDtypeStruct((NGRAPH, 2), jnp.float32),
    )(*args)


def kernel(rna_ids, rna_feat, atac_ids, atac_feat, chrom_mask,
           rna_batch, atac_batch, params):
    p = params
    emb_r = p["rna_emb"][rna_ids]
    emb_a = p["atac_emb"][atac_ids]
    mask2d = chrom_mask[..., 0]

    q3, rv3, rself = _proj_side(
        emb_r, rna_feat, p["rna_query_w"], p["rna_query_b"],
        p["rna_value_w"], p["rna_value_b"],
        p["rna_self_w"], p["rna_self_b"], bm=1024)
    k3, av3, aself = _proj_side(
        emb_a, atac_feat, p["atac_key_w"], p["atac_key_b"],
        p["atac_value_w"], p["atac_value_b"],
        p["atac_self_w"], p["atac_self_b"], bm=1024)

    r2a3, a2r3 = _attn(q3, k3, av3, rv3, mask2d)

    sum_r, cnt_r = _reduce_side(r2a3, rself, rna_batch,
                                p["rna_out_w"], p["rna_out_b"],
                                p["red_rna_w"], p["red_rna_b"], bm=1024)
    sum_a, cnt_a = _reduce_side(a2r3, aself, atac_batch,
                                p["atac_out_w"], p["atac_out_b"],
                                p["red_atac_w"], p["red_atac_b"], bm=1024)
    return _head_mlp(sum_r, cnt_r, sum_a, cnt_a, p)
